# Initial kernel scaffold; baseline (speedup 1.0000x reference)
import jax
import jax.numpy as jnp
from jax.experimental import pallas as pl


def kernel(s, kernel):
    raise NotImplementedError("write your pallas kernel here")


# speedup vs baseline: 1.0043x; 1.0043x over previous
import functools

import jax
import jax.numpy as jnp
from jax import lax
from jax.experimental import pallas as pl
from jax.experimental.pallas import tpu as pltpu
from jax.experimental.pallas import tpu_sc as plsc

L = 20
B = 16384
NC = 2
NS = 16
LANES = 16
NW = NC * NS
BPW = B // NW
NV = BPW // LANES

_LN2 = 0.6931471805599453
_SQRT2 = 1.4142135623730951

_LOG_COEFFS = (
    7.0376836292e-2, -1.1514610310e-1, 1.1676998740e-1, -1.2420140846e-1,
    1.4249322787e-1, -1.6668057665e-1, 2.0000714765e-1, -2.4999993993e-1,
    3.3333331174e-1,
)


def _log16(x):
    bits = lax.bitcast_convert_type(x, jnp.int32)
    e = lax.shift_right_logical(bits, 23) - 127
    m = lax.bitcast_convert_type((bits & 0x7FFFFF) | 0x3F800000, jnp.float32)
    big = m > _SQRT2
    m = jnp.where(big, m * 0.5, m)
    e = jnp.where(big, e + 1, e)
    f = m - 1.0
    z = f * f
    p = jnp.full((LANES,), _LOG_COEFFS[0], jnp.float32)
    for c in _LOG_COEFFS[1:]:
        p = p * f + c
    y = f * z * p - 0.5 * z
    return (f + y) + e.astype(jnp.float32) * _LN2


def _sc_body(s_hbm, table_hbm, out_hbm, s_v, idx_v, val_v, sem):
    wid = lax.axis_index("s") * NC + lax.axis_index("c")
    base = wid * BPW

    pltpu.sync_copy(s_hbm.at[:, pl.ds(base, BPW)], s_v)

    def compute_idx(v, carry):
        off = v * LANES
        acc = s_v[0, pl.ds(off, LANES)]
        for l in range(1, L):
            acc = acc | lax.shift_left(s_v[l, pl.ds(off, LANES)], l)
        idx_v[pl.ds(off, LANES)] = acc
        return carry

    lax.fori_loop(0, NV, compute_idx, 0)

    pltpu.async_copy(table_hbm.at[idx_v], val_v, sem).wait()

    def do_log(v, carry):
        off = v * LANES
        val_v[pl.ds(off, LANES)] = _log16(val_v[pl.ds(off, LANES)])
        return carry

    lax.fori_loop(0, NV, do_log, 0)

    pltpu.sync_copy(val_v, out_hbm.at[pl.ds(base, BPW)])


_sc_call = pl.kernel(
    _sc_body,
    out_type=jax.ShapeDtypeStruct((B,), jnp.float32),
    mesh=plsc.VectorSubcoreMesh(core_axis_name="c", subcore_axis_name="s"),
    scratch_types=[
        pltpu.VMEM((L, BPW), jnp.int32),
        pltpu.VMEM((BPW,), jnp.int32),
        pltpu.VMEM((BPW,), jnp.float32),
        pltpu.SemaphoreType.DMA,
    ],
)


def kernel(s, table):
    return _sc_call(s.astype(jnp.int32), table)
